# Initial kernel scaffold; baseline (speedup 1.0000x reference)
import functools

import jax
import jax.numpy as jnp
from jax import lax
from jax.experimental import pallas as pl
from jax.experimental.pallas import tpu as pltpu
from jax.experimental.pallas import tpu_sc as plsc

B = 16384
D = 32
NC, NS = 2, 16
NW = NC * NS
PER_W = B // NW

_mesh = plsc.VectorSubcoreMesh(core_axis_name="c", subcore_axis_name="s")


@functools.partial(
    pl.kernel,
    out_type=[
        jax.ShapeDtypeStruct((2, B, D), jnp.float32),
        jax.ShapeDtypeStruct((4, B, D), jnp.float32),
    ],
    mesh=_mesh,
    scratch_types=[
        pltpu.VMEM((PER_W,), jnp.int32),
        pltpu.VMEM((PER_W, D), jnp.float32),
        pltpu.SemaphoreType.DMA,
    ],
)
def _sc_gather(t_tab, a_tab, t_idx, a_idx, t_out, a_out, idx_v, rows_v, sem):
    wid = lax.axis_index("s") * NC + lax.axis_index("c")
    base = wid * PER_W
    for j in range(2):
        pltpu.sync_copy(t_idx.at[j].at[pl.ds(base, PER_W)], idx_v)
        pltpu.async_copy(t_tab.at[idx_v], rows_v, sem).wait()
        pltpu.sync_copy(rows_v, t_out.at[j].at[pl.ds(base, PER_W)])
    for j in range(4):
        pltpu.sync_copy(a_idx.at[j].at[pl.ds(base, PER_W)], idx_v)
        pltpu.async_copy(a_tab.at[idx_v], rows_v, sem).wait()
        pltpu.sync_copy(rows_v, a_out.at[j].at[pl.ds(base, PER_W)])


NB = 2048


def _combine_body(t_ref, a_ref, w_ref, b_ref, o_ref):
    acc = jnp.dot(t_ref[0], w_ref[0:D, :], preferred_element_type=jnp.float32)
    acc = acc + jnp.dot(t_ref[1], w_ref[D:2 * D, :],
                        preferred_element_type=jnp.float32)
    for j in range(4):
        acc = acc + jnp.dot(a_ref[j], w_ref[(2 + j) * D:(3 + j) * D, :],
                            preferred_element_type=jnp.float32)
    o_ref[...] = acc + b_ref[...]


def _combine(t_emb, a_emb, wt, b2):
    return pl.pallas_call(
        _combine_body,
        grid=(B // NB,),
        in_specs=[
            pl.BlockSpec((2, NB, D), lambda i: (0, i, 0)),
            pl.BlockSpec((4, NB, D), lambda i: (0, i, 0)),
            pl.BlockSpec((6 * D, D), lambda i: (0, 0)),
            pl.BlockSpec((1, D), lambda i: (0, 0)),
        ],
        out_specs=pl.BlockSpec((NB, D), lambda i: (i, 0)),
        out_shape=jax.ShapeDtypeStruct((B, D), jnp.float32),
    )(t_emb, a_emb, wt, b2)


def kernel(type_ids, ability_ids, type_table, ability_table, W, b):
    t_idx = type_ids.T.astype(jnp.int32)
    a_idx = ability_ids.T.astype(jnp.int32)
    t_emb, a_emb = _sc_gather(type_table, ability_table, t_idx, a_idx)
    wt = W.T
    b2 = b.reshape(1, D)
    return _combine(t_emb, a_emb, wt, b2)


# speedup vs baseline: 2.6647x; 2.6647x over previous
import functools

import jax
import jax.numpy as jnp
from jax import lax
from jax.experimental import pallas as pl
from jax.experimental.pallas import tpu as pltpu
from jax.experimental.pallas import tpu_sc as plsc

B = 16384
D = 32
NC, NS = 2, 16
NW = NC * NS
PER_W = B // NW

_mesh = plsc.VectorSubcoreMesh(core_axis_name="c", subcore_axis_name="s")


@functools.partial(
    pl.kernel,
    out_type=[
        jax.ShapeDtypeStruct((2, B, D), jnp.float32),
        jax.ShapeDtypeStruct((4, B, D), jnp.float32),
    ],
    mesh=_mesh,
    scratch_types=[
        pltpu.VMEM((PER_W,), jnp.int32),
        pltpu.VMEM((PER_W, D), jnp.float32),
        pltpu.SemaphoreType.DMA,
    ],
    compiler_params=pltpu.CompilerParams(use_tc_tiling_on_sc=False),
)
def _sc_gather(t_tab, a_tab, t_idx, a_idx, t_out, a_out, idx_v, rows_v, sem):
    wid = lax.axis_index("s") * NC + lax.axis_index("c")
    base = wid * PER_W
    for j in range(2):
        pltpu.sync_copy(t_idx.at[j].at[pl.ds(base, PER_W)], idx_v)
        pltpu.async_copy(t_tab.at[idx_v], rows_v, sem).wait()
        pltpu.sync_copy(rows_v, t_out.at[j].at[pl.ds(base, PER_W)])
    for j in range(4):
        pltpu.sync_copy(a_idx.at[j].at[pl.ds(base, PER_W)], idx_v)
        pltpu.async_copy(a_tab.at[idx_v], rows_v, sem).wait()
        pltpu.sync_copy(rows_v, a_out.at[j].at[pl.ds(base, PER_W)])


NB = 2048


def _combine_body(t_ref, a_ref, w_ref, b_ref, o_ref):
    acc = jnp.dot(t_ref[0], w_ref[0:D, :], preferred_element_type=jnp.float32)
    acc = acc + jnp.dot(t_ref[1], w_ref[D:2 * D, :],
                        preferred_element_type=jnp.float32)
    for j in range(4):
        acc = acc + jnp.dot(a_ref[j], w_ref[(2 + j) * D:(3 + j) * D, :],
                            preferred_element_type=jnp.float32)
    o_ref[...] = acc + b_ref[...]


def _combine(t_emb, a_emb, wt, b2):
    return pl.pallas_call(
        _combine_body,
        grid=(B // NB,),
        in_specs=[
            pl.BlockSpec((2, NB, D), lambda i: (0, i, 0)),
            pl.BlockSpec((4, NB, D), lambda i: (0, i, 0)),
            pl.BlockSpec((6 * D, D), lambda i: (0, 0)),
            pl.BlockSpec((1, D), lambda i: (0, 0)),
        ],
        out_specs=pl.BlockSpec((NB, D), lambda i: (i, 0)),
        out_shape=jax.ShapeDtypeStruct((B, D), jnp.float32),
    )(t_emb, a_emb, wt, b2)


def kernel(type_ids, ability_ids, type_table, ability_table, W, b):
    t_idx = type_ids.T.astype(jnp.int32)
    a_idx = ability_ids.T.astype(jnp.int32)
    t_emb, a_emb = _sc_gather(type_table, ability_table, t_idx, a_idx)
    wt = W.T
    b2 = b.reshape(1, D)
    return _combine(t_emb, a_emb, wt, b2)
